# baseline (speedup 1.0000x reference)
import functools

import jax
import jax.numpy as jnp
from jax import lax
from jax.experimental import pallas as pl
from jax.experimental.pallas import tpu as pltpu
from jax.experimental.pallas import tpu_sc as plsc

N = 100000
D = 64
MAX_DEPTH = 20
BLK = 128
NBLK = (N + BLK - 1) // BLK

_info = plsc.get_sparse_core_info()
NC, NS = _info.num_cores, _info.num_subcores
NW = NC * NS
STEPS = (NBLK + NW - 1) // NW

_mesh = plsc.VectorSubcoreMesh(core_axis_name="c", subcore_axis_name="s")


@functools.partial(
    pl.kernel,
    mesh=_mesh,
    out_type=jax.ShapeDtypeStruct((N, D), jnp.float32),
    compiler_params=pltpu.CompilerParams(use_tc_tiling_on_sc=False),
    scratch_types=[
        pltpu.VMEM((BLK,), jnp.int32),
        pltpu.VMEM((BLK,), jnp.int32),
        pltpu.VMEM((BLK,), jnp.int32),
        pltpu.VMEM((BLK, D), jnp.float32),
        pltpu.VMEM((BLK, D), jnp.float32),
        pltpu.VMEM((BLK, D), jnp.float32),
        pltpu.SemaphoreType.DMA,
    ],
)
def _encode(x0_hbm, x1_hbm, dep_hbm, ttab, atab, dtab, out_hbm,
            idx0_v, idx1_v, dep_v, t_v, a_v, d_v, sem):
    wid = lax.axis_index("s") * NC + lax.axis_index("c")

    def step(j, carry):
        blk = wid + j * NW

        @pl.when(blk < NBLK)
        def _():
            base = lax.min(blk * BLK, N - BLK)
            c0 = pltpu.async_copy(x0_hbm.at[pl.ds(base, BLK)], idx0_v, sem)
            c1 = pltpu.async_copy(x1_hbm.at[pl.ds(base, BLK)], idx1_v, sem)
            c2 = pltpu.async_copy(dep_hbm.at[pl.ds(base, BLK)], dep_v, sem)
            c0.wait()
            c1.wait()
            c2.wait()
            for i in range(BLK // 16):
                s = pl.ds(i * 16, 16)
                dep_v[s] = jnp.minimum(dep_v[s], MAX_DEPTH)
            g0 = pltpu.async_copy(ttab.at[idx0_v], t_v, sem)
            g1 = pltpu.async_copy(atab.at[idx1_v], a_v, sem)
            g2 = pltpu.async_copy(dtab.at[dep_v], d_v, sem)
            g0.wait()
            g1.wait()
            g2.wait()

            def row(r, rc):
                for c in range(D // 16):
                    s = pl.ds(c * 16, 16)
                    a_v[r, s] = a_v[r, s] + t_v[r, s] + d_v[r, s]
                return rc

            lax.fori_loop(0, BLK, row, 0)
            pltpu.sync_copy(a_v, out_hbm.at[pl.ds(base, BLK)])

        return carry

    lax.fori_loop(0, STEPS, step, 0)


def kernel(x, depth, type_table, attribute_table, depth_table):
    x0 = x[:, 0]
    x1 = x[:, 1]
    return _encode(x0, x1, depth, type_table, attribute_table, depth_table)


# speedup vs baseline: 1.0009x; 1.0009x over previous
import functools

import jax
import jax.numpy as jnp
from jax import lax
from jax.experimental import pallas as pl
from jax.experimental.pallas import tpu as pltpu
from jax.experimental.pallas import tpu_sc as plsc

N = 100000
D = 64
MAX_DEPTH = 20
BLK = 128
NBLK = (N + BLK - 1) // BLK
NBUF = 3

_info = plsc.get_sparse_core_info()
NC, NS = _info.num_cores, _info.num_subcores
NW = NC * NS

BASE_BLKS = NBLK // NW
EXTRA = NBLK - BASE_BLKS * NW
MAX_BLKS = BASE_BLKS + 1
WIN = MAX_BLKS * BLK

_mesh = plsc.VectorSubcoreMesh(core_axis_name="c", subcore_axis_name="s")


@functools.partial(
    pl.kernel,
    mesh=_mesh,
    out_type=jax.ShapeDtypeStruct((N, D), jnp.float32),
    compiler_params=pltpu.CompilerParams(use_tc_tiling_on_sc=False),
    scratch_types=[
        pltpu.VMEM((WIN,), jnp.int32),
        pltpu.VMEM((WIN,), jnp.int32),
        pltpu.VMEM((WIN,), jnp.int32),
    ]
    + [pltpu.VMEM((BLK, D), jnp.float32)] * (3 * NBUF)
    + [pltpu.SemaphoreType.DMA] * (1 + 2 * NBUF),
)
def _encode(x0_hbm, x1_hbm, dep_hbm, ttab, atab, dtab, out_hbm,
            idx0_v, idx1_v, dep_v,
            t0, t1, t2, a0, a1, a2, d0, d1, d2,
            ssem, g0sem, g1sem, g2sem, w0sem, w1sem, w2sem):
    t_bufs = (t0, t1, t2)
    a_bufs = (a0, a1, a2)
    d_bufs = (d0, d1, d2)
    gsems = (g0sem, g1sem, g2sem)
    wsems = (w0sem, w1sem, w2sem)

    wid = lax.axis_index("s") * NC + lax.axis_index("c")
    first_blk = wid * BASE_BLKS + lax.min(wid, EXTRA)
    n_blk = BASE_BLKS + jnp.where(wid < EXTRA, 1, 0)
    start = lax.min(first_blk * BLK, N - WIN)

    c0 = pltpu.async_copy(x0_hbm.at[pl.ds(start, WIN)], idx0_v, ssem)
    c1 = pltpu.async_copy(x1_hbm.at[pl.ds(start, WIN)], idx1_v, ssem)
    c2 = pltpu.async_copy(dep_hbm.at[pl.ds(start, WIN)], dep_v, ssem)
    c0.wait()
    c1.wait()
    c2.wait()

    def clamp(i, carry):
        s = pl.ds(i * 16, 16)
        dep_v[s] = jnp.minimum(dep_v[s], MAX_DEPTH)
        return carry

    lax.fori_loop(0, WIN // 16, clamp, 0)

    def lbase(k):
        return lax.min((first_blk + k) * BLK, N - BLK) - start

    def issue(k, b):
        lb = lbase(k)
        pltpu.async_copy(ttab.at[idx0_v.at[pl.ds(lb, BLK)]], t_bufs[b], gsems[b])
        pltpu.async_copy(atab.at[idx1_v.at[pl.ds(lb, BLK)]], a_bufs[b], gsems[b])
        pltpu.async_copy(dtab.at[dep_v.at[pl.ds(lb, BLK)]], d_bufs[b], gsems[b])

    issue(0, 0)

    for k in range(MAX_BLKS):
        b = k % NBUF
        nb = (k + 1) % NBUF
        if k + 1 < MAX_BLKS:
            def prefetch():
                if k + 1 >= NBUF:
                    pltpu.make_async_copy(
                        a_bufs[nb], out_hbm.at[pl.ds(0, BLK)], wsems[nb]).wait()
                issue(k + 1, nb)

            if k + 1 < BASE_BLKS:
                prefetch()
            else:
                pl.when(k + 1 < n_blk)(prefetch)

        def compute():
            pltpu.make_async_copy(
                ttab.at[idx0_v.at[pl.ds(0, BLK)]], t_bufs[b], gsems[b]).wait()
            pltpu.make_async_copy(
                atab.at[idx1_v.at[pl.ds(0, BLK)]], a_bufs[b], gsems[b]).wait()
            pltpu.make_async_copy(
                dtab.at[dep_v.at[pl.ds(0, BLK)]], d_bufs[b], gsems[b]).wait()

            def row(r, carry):
                for c in range(D // 16):
                    s = pl.ds(c * 16, 16)
                    a_bufs[b][r, s] = a_bufs[b][r, s] + t_bufs[b][r, s] + d_bufs[b][r, s]
                return carry

            lax.fori_loop(0, BLK, row, 0)
            gb = lax.min((first_blk + k) * BLK, N - BLK)
            pltpu.async_copy(a_bufs[b], out_hbm.at[pl.ds(gb, BLK)], wsems[b])

        if k < BASE_BLKS:
            compute()
        else:
            pl.when(k < n_blk)(compute)

    for i in range(NBUF):
        pltpu.make_async_copy(
            a_bufs[i], out_hbm.at[pl.ds(0, BLK)], wsems[i]).wait()


def kernel(x, depth, type_table, attribute_table, depth_table):
    x0 = x[:, 0]
    x1 = x[:, 1]
    return _encode(x0, x1, depth, type_table, attribute_table, depth_table)
